# baseline (device time: 116203 ns/iter reference)
import jax
import jax.numpy as jnp
from jax import lax
from jax.experimental import pallas as pl
from jax.experimental.pallas import tpu as pltpu

N_DEV = 4


def kernel(x, w_mat, scale_x, scale_w):
    m_per, k = x.shape
    n_total = w_mat.shape[1]
    n_per = n_total // N_DEV
    H = m_per // 2
    Q = H // 2
    KS = 512
    n_strips = k // KS

    x = x.astype(jnp.float8_e5m2)

    def body(x_ref, w_ref, sx_ref, sw_ref, out_ref, rbuf, hbuf, w8,
             wstrip, stg, ssems, rsems, wsems, osems):
        my = lax.axis_index("i")
        left = (my - 1) % N_DEV
        right = (my + 1) % N_DEV
        opp = (my + 2) % N_DEV
        scale = sx_ref[0] * sw_ref[0]

        def w_dma(j):
            d = pltpu.make_async_copy(
                w_ref.at[pl.ds(j * KS, KS), pl.ds(my * n_per, n_per)],
                wstrip.at[j % 2],
                wsems.at[j % 2],
            )
            d.start()
            return d

        wd = {j: w_dma(j) for j in range(2)}

        barrier_sem = pltpu.get_barrier_semaphore()
        for nbr in (left, right):
            pl.semaphore_signal(
                barrier_sem, inc=1,
                device_id=(nbr,), device_id_type=pl.DeviceIdType.MESH,
            )
        pl.semaphore_wait(barrier_sem, 2)

        def rc(src, dst, sem_i, dev):
            return pltpu.make_async_remote_copy(
                src_ref=src, dst_ref=dst,
                send_sem=ssems.at[sem_i], recv_sem=rsems.at[sem_i],
                device_id=(dev,), device_id_type=pl.DeviceIdType.MESH,
            )

        p1r = [rc(x_ref.at[pl.ds(h * H, H)], rbuf.at[0, pl.ds(h * H, H)],
                  h, right) for h in range(2)]
        p1l = [rc(x_ref.at[pl.ds(h * H, H)], rbuf.at[1, pl.ds(h * H, H)],
                  2 + h, left) for h in range(2)]
        p2r = [rc(rbuf.at[0, pl.ds(q * Q, Q)], hbuf.at[0, pl.ds(q * Q, Q)],
                  4 + q, right) for q in range(2)]
        p2l = [rc(rbuf.at[1, pl.ds(H + q * Q, Q)],
                  hbuf.at[1, pl.ds(q * Q, Q)], 6 + q, left)
               for q in range(2)]

        for r in p1r + p1l:
            r.start()

        for j in range(n_strips):
            wd[j].wait()
            w8[pl.ds(j * KS, KS), :] = wstrip[j % 2].astype(jnp.float8_e5m2)
            if j + 2 < n_strips:
                wd[j + 2] = w_dma(j + 2)

        out_dmas = []

        def gemm(src_ref, src_off, out_off, rows):
            i = len(out_dmas)
            slot = i % 2
            if i >= 2:
                out_dmas[i - 2].wait()
            acc = jnp.dot(src_ref[pl.ds(src_off, rows)], w8[...],
                          preferred_element_type=jnp.float32)
            stg[slot, pl.ds(0, rows), :] = acc * scale
            d = pltpu.make_async_copy(
                stg.at[slot, pl.ds(0, rows)],
                out_ref.at[pl.ds(out_off, rows)],
                osems.at[slot],
            )
            d.start()
            out_dmas.append(d)

        gemm(x_ref, 0, my * m_per, H)
        gemm(x_ref, H, my * m_per + H, H)

        p1r[0].wait_recv()
        p2r[0].start()
        p2r[1].start()
        gemm(rbuf.at[0], 0, left * m_per, H)

        p1l[0].wait_recv()
        gemm(rbuf.at[1], 0, right * m_per, H)

        p1r[1].wait_recv()
        gemm(rbuf.at[0], H, left * m_per + H, H)

        p1l[1].wait_recv()
        p2l[0].start()
        p2l[1].start()
        gemm(rbuf.at[1], H, right * m_per + H, H)

        p2r[0].wait_recv()
        gemm(hbuf.at[0], 0, opp * m_per, Q)
        p2l[0].wait_recv()
        gemm(hbuf.at[1], 0, opp * m_per + H, Q)
        p2r[1].wait_recv()
        gemm(hbuf.at[0], Q, opp * m_per + Q, Q)
        p2l[1].wait_recv()
        gemm(hbuf.at[1], Q, opp * m_per + H + Q, Q)

        out_dmas[-2].wait()
        out_dmas[-1].wait()
        for r in p1r + p1l + p2r + p2l:
            r.wait_send()

    return pl.pallas_call(
        body,
        out_shape=jax.ShapeDtypeStruct((N_DEV * m_per, n_per), jnp.float32),
        in_specs=[
            pl.BlockSpec(memory_space=pltpu.VMEM),
            pl.BlockSpec(memory_space=pl.ANY),
            pl.BlockSpec(memory_space=pltpu.SMEM),
            pl.BlockSpec(memory_space=pltpu.SMEM),
        ],
        out_specs=pl.BlockSpec(memory_space=pl.ANY),
        scratch_shapes=[
            pltpu.VMEM((2, m_per, k), x.dtype),
            pltpu.VMEM((2, H, k), x.dtype),
            pltpu.VMEM((k, n_per), x.dtype),
            pltpu.VMEM((2, KS, n_per), jnp.float32),
            pltpu.VMEM((2, H, n_per), jnp.float32),
            pltpu.SemaphoreType.DMA((8,)),
            pltpu.SemaphoreType.DMA((8,)),
            pltpu.SemaphoreType.DMA((2,)),
            pltpu.SemaphoreType.DMA((2,)),
        ],
        compiler_params=pltpu.CompilerParams(
            collective_id=0,
            vmem_limit_bytes=64 * 1024 * 1024,
        ),
    )(x, w_mat, scale_x, scale_w)


# device time: 106136 ns/iter; 1.0949x vs baseline; 1.0949x over previous
import jax
import jax.numpy as jnp
from jax import lax
from jax.experimental import pallas as pl
from jax.experimental.pallas import tpu as pltpu

N_DEV = 4


def kernel(x, w_mat, scale_x, scale_w):
    m_per, k = x.shape
    n_total = w_mat.shape[1]
    n_per = n_total // N_DEV
    H = m_per // 2
    QS = [(0, 320), (320, 192)]
    KS = 512
    n_strips = k // KS

    def body(x_ref, w_ref, sx_ref, sw_ref, out_ref, rbuf, hbuf, x8,
             xstrip, w8, wstrip, stg, ssems, rsems, xsems, wsems, osems):
        my = lax.axis_index("i")
        left = (my - 1) % N_DEV
        right = (my + 1) % N_DEV
        opp = (my + 2) % N_DEV
        scale = sx_ref[0] * sw_ref[0]

        def x_dma(h):
            d = pltpu.make_async_copy(
                x_ref.at[pl.ds(h * H, H)], xstrip, xsems.at[0])
            d.start()
            return d

        xd = x_dma(0)

        def w_dma(j):
            d = pltpu.make_async_copy(
                w_ref.at[pl.ds(j * KS, KS), pl.ds(my * n_per, n_per)],
                wstrip.at[j % 2],
                wsems.at[j % 2],
            )
            d.start()
            return d

        wd = {j: w_dma(j) for j in range(2)}

        barrier_sem = pltpu.get_barrier_semaphore()
        for nbr in (left, right):
            pl.semaphore_signal(
                barrier_sem, inc=1,
                device_id=(nbr,), device_id_type=pl.DeviceIdType.MESH,
            )
        pl.semaphore_wait(barrier_sem, 2)

        def rc(src, dst, sem_i, dev):
            return pltpu.make_async_remote_copy(
                src_ref=src, dst_ref=dst,
                send_sem=ssems.at[sem_i], recv_sem=rsems.at[sem_i],
                device_id=(dev,), device_id_type=pl.DeviceIdType.MESH,
            )

        p1r = [rc(x8.at[pl.ds(h * H, H)], rbuf.at[0, pl.ds(h * H, H)],
                  h, right) for h in range(2)]
        p1l = [rc(x8.at[pl.ds(h * H, H)], rbuf.at[1, pl.ds(h * H, H)],
                  2 + h, left) for h in range(2)]
        p2r = [rc(rbuf.at[0, pl.ds(o, r)], hbuf.at[0, pl.ds(o, r)],
                  4 + i, right) for i, (o, r) in enumerate(QS)]
        p2l = [rc(rbuf.at[1, pl.ds(H + o, r)], hbuf.at[1, pl.ds(o, r)],
                  6 + i, left) for i, (o, r) in enumerate(QS)]

        xd.wait()
        x8[pl.ds(0, H), :] = xstrip[...].astype(jnp.float8_e5m2)
        p1r[0].start()
        p1l[0].start()
        xd = x_dma(1)
        xd.wait()
        x8[pl.ds(H, H), :] = xstrip[...].astype(jnp.float8_e5m2)
        p1r[1].start()
        p1l[1].start()

        for j in range(n_strips):
            wd[j].wait()
            w8[pl.ds(j * KS, KS), :] = wstrip[j % 2].astype(jnp.float8_e5m2)
            if j + 2 < n_strips:
                wd[j + 2] = w_dma(j + 2)

        out_dmas = []

        def gemm(src_ref, src_off, out_off, rows):
            i = len(out_dmas)
            slot = i % 2
            if i >= 2:
                out_dmas[i - 2].wait()
            acc = jnp.dot(src_ref[pl.ds(src_off, rows)], w8[...],
                          preferred_element_type=jnp.float32)
            stg[slot, pl.ds(0, rows), :] = acc * scale
            d = pltpu.make_async_copy(
                stg.at[slot, pl.ds(0, rows)],
                out_ref.at[pl.ds(out_off, rows)],
                osems.at[slot],
            )
            d.start()
            out_dmas.append(d)

        gemm(x8, 0, my * m_per, H)
        gemm(x8, H, my * m_per + H, H)

        p1r[0].wait_recv()
        p2r[0].start()
        p2r[1].start()
        gemm(rbuf.at[0], 0, left * m_per, H)

        p1l[0].wait_recv()
        gemm(rbuf.at[1], 0, right * m_per, H)

        p1r[1].wait_recv()
        gemm(rbuf.at[0], H, left * m_per + H, H)

        p1l[1].wait_recv()
        p2l[0].start()
        p2l[1].start()
        gemm(rbuf.at[1], H, right * m_per + H, H)

        p2r[0].wait_recv()
        gemm(hbuf.at[0], 0, opp * m_per, QS[0][1])
        p2l[0].wait_recv()
        gemm(hbuf.at[1], 0, opp * m_per + H, QS[0][1])
        p2r[1].wait_recv()
        gemm(hbuf.at[0], QS[1][0], opp * m_per + QS[1][0], QS[1][1])
        p2l[1].wait_recv()
        gemm(hbuf.at[1], QS[1][0], opp * m_per + H + QS[1][0], QS[1][1])

        out_dmas[-2].wait()
        out_dmas[-1].wait()
        for r in p1r + p1l + p2r + p2l:
            r.wait_send()

    return pl.pallas_call(
        body,
        out_shape=jax.ShapeDtypeStruct((N_DEV * m_per, n_per), jnp.float32),
        in_specs=[
            pl.BlockSpec(memory_space=pl.ANY),
            pl.BlockSpec(memory_space=pl.ANY),
            pl.BlockSpec(memory_space=pltpu.SMEM),
            pl.BlockSpec(memory_space=pltpu.SMEM),
        ],
        out_specs=pl.BlockSpec(memory_space=pl.ANY),
        scratch_shapes=[
            pltpu.VMEM((2, m_per, k), jnp.float8_e5m2),
            pltpu.VMEM((2, H, k), jnp.float8_e5m2),
            pltpu.VMEM((m_per, k), jnp.float8_e5m2),
            pltpu.VMEM((H, k), jnp.float32),
            pltpu.VMEM((k, n_per), jnp.float8_e5m2),
            pltpu.VMEM((2, KS, n_per), jnp.float32),
            pltpu.VMEM((2, H, n_per), jnp.float32),
            pltpu.SemaphoreType.DMA((8,)),
            pltpu.SemaphoreType.DMA((8,)),
            pltpu.SemaphoreType.DMA((1,)),
            pltpu.SemaphoreType.DMA((2,)),
            pltpu.SemaphoreType.DMA((2,)),
        ],
        compiler_params=pltpu.CompilerParams(
            collective_id=0,
            vmem_limit_bytes=64 * 1024 * 1024,
        ),
    )(x, w_mat, scale_x, scale_w)


# device time: 105838 ns/iter; 1.0979x vs baseline; 1.0028x over previous
import jax
import jax.numpy as jnp
from jax import lax
from jax.experimental import pallas as pl
from jax.experimental.pallas import tpu as pltpu

N_DEV = 4


def kernel(x, w_mat, scale_x, scale_w):
    m_per, k = x.shape
    n_total = w_mat.shape[1]
    n_per = n_total // N_DEV
    H = m_per // 2
    QS = [(0, 320), (320, 192)]
    KS = 1024
    n_strips = 2 * (k // KS)

    def body(x_ref, w_ref, sx_ref, sw_ref, out_ref, rbuf, hbuf, x8,
             xstrip, w8, wstrip, stg, ssems, rsems, xsems, wsems, osems):
        my = lax.axis_index("i")
        left = (my - 1) % N_DEV
        right = (my + 1) % N_DEV
        opp = (my + 2) % N_DEV
        scale = sx_ref[0] * sw_ref[0]

        def x_dma(h):
            d = pltpu.make_async_copy(
                x_ref.at[pl.ds(h * H, H)], xstrip, xsems.at[0])
            d.start()
            return d

        xd = x_dma(0)

        NH = n_per // 2

        def w_dma(j):
            d = pltpu.make_async_copy(
                w_ref.at[pl.ds((j % 4) * KS, KS),
                         pl.ds(my * n_per + (j // 4) * NH, NH)],
                wstrip.at[j % 2],
                wsems.at[j % 2],
            )
            d.start()
            return d

        wd = {j: w_dma(j) for j in range(2)}

        barrier_sem = pltpu.get_barrier_semaphore()
        for nbr in (left, right):
            pl.semaphore_signal(
                barrier_sem, inc=1,
                device_id=(nbr,), device_id_type=pl.DeviceIdType.MESH,
            )
        pl.semaphore_wait(barrier_sem, 2)

        def rc(src, dst, sem_i, dev):
            return pltpu.make_async_remote_copy(
                src_ref=src, dst_ref=dst,
                send_sem=ssems.at[sem_i], recv_sem=rsems.at[sem_i],
                device_id=(dev,), device_id_type=pl.DeviceIdType.MESH,
            )

        p1r = [rc(x8.at[pl.ds(h * H, H)], rbuf.at[0, pl.ds(h * H, H)],
                  h, right) for h in range(2)]
        p1l = [rc(x8.at[pl.ds(h * H, H)], rbuf.at[1, pl.ds(h * H, H)],
                  2 + h, left) for h in range(2)]
        p2r = [rc(rbuf.at[0, pl.ds(o, r)], hbuf.at[0, pl.ds(o, r)],
                  4 + i, right) for i, (o, r) in enumerate(QS)]
        p2l = [rc(rbuf.at[1, pl.ds(H + o, r)], hbuf.at[1, pl.ds(o, r)],
                  6 + i, left) for i, (o, r) in enumerate(QS)]

        xd.wait()
        x8[pl.ds(0, H), :] = xstrip[...].astype(jnp.float8_e5m2)
        p1r[0].start()
        p1l[0].start()
        xd = x_dma(1)
        xd.wait()
        x8[pl.ds(H, H), :] = xstrip[...].astype(jnp.float8_e5m2)
        p1r[1].start()
        p1l[1].start()

        out_dmas = []

        def gemm(src_ref, src_off, out_off, rows, n0=0, nc=n_per):
            i = len(out_dmas)
            slot = i % 2
            if i >= 2:
                out_dmas[i - 2].wait()
            acc = jnp.dot(src_ref[pl.ds(src_off, rows)],
                          w8[:, pl.ds(n0, nc)],
                          preferred_element_type=jnp.float32)
            stg[slot, pl.ds(0, rows), pl.ds(n0, nc)] = acc * scale
            d = pltpu.make_async_copy(
                stg.at[slot, pl.ds(0, rows), pl.ds(n0, nc)],
                out_ref.at[pl.ds(out_off, rows), pl.ds(n0, nc)],
                osems.at[slot],
            )
            d.start()
            out_dmas.append(d)

        for j in range(n_strips):
            wd[j].wait()
            w8[pl.ds((j % 4) * KS, KS), pl.ds((j // 4) * NH, NH)] = (
                wstrip[j % 2].astype(jnp.float8_e5m2))
            if j + 2 < n_strips:
                wd[j + 2] = w_dma(j + 2)
            if j == 3:
                gemm(x8, 0, my * m_per, H, 0, NH)
                gemm(x8, H, my * m_per + H, H, 0, NH)

        gemm(x8, 0, my * m_per, H, NH, NH)
        gemm(x8, H, my * m_per + H, H, NH, NH)

        p1r[0].wait_recv()
        p2r[0].start()
        p2r[1].start()
        gemm(rbuf.at[0], 0, left * m_per, H)

        p1l[0].wait_recv()
        gemm(rbuf.at[1], 0, right * m_per, H)

        p1r[1].wait_recv()
        gemm(rbuf.at[0], H, left * m_per + H, H)

        p1l[1].wait_recv()
        p2l[0].start()
        p2l[1].start()
        gemm(rbuf.at[1], H, right * m_per + H, H)

        p2r[0].wait_recv()
        gemm(hbuf.at[0], 0, opp * m_per, QS[0][1])
        p2l[0].wait_recv()
        gemm(hbuf.at[1], 0, opp * m_per + H, QS[0][1])
        p2r[1].wait_recv()
        gemm(hbuf.at[0], QS[1][0], opp * m_per + QS[1][0], QS[1][1])
        p2l[1].wait_recv()
        gemm(hbuf.at[1], QS[1][0], opp * m_per + H + QS[1][0], QS[1][1])

        out_dmas[-2].wait()
        out_dmas[-1].wait()
        for r in p1r + p1l + p2r + p2l:
            r.wait_send()

    return pl.pallas_call(
        body,
        out_shape=jax.ShapeDtypeStruct((N_DEV * m_per, n_per), jnp.float32),
        in_specs=[
            pl.BlockSpec(memory_space=pl.ANY),
            pl.BlockSpec(memory_space=pl.ANY),
            pl.BlockSpec(memory_space=pltpu.SMEM),
            pl.BlockSpec(memory_space=pltpu.SMEM),
        ],
        out_specs=pl.BlockSpec(memory_space=pl.ANY),
        scratch_shapes=[
            pltpu.VMEM((2, m_per, k), jnp.float8_e5m2),
            pltpu.VMEM((2, H, k), jnp.float8_e5m2),
            pltpu.VMEM((m_per, k), jnp.float8_e5m2),
            pltpu.VMEM((H, k), jnp.float32),
            pltpu.VMEM((k, n_per), jnp.float8_e5m2),
            pltpu.VMEM((2, KS, n_per // 2), jnp.float32),
            pltpu.VMEM((2, H, n_per), jnp.float32),
            pltpu.SemaphoreType.DMA((8,)),
            pltpu.SemaphoreType.DMA((8,)),
            pltpu.SemaphoreType.DMA((1,)),
            pltpu.SemaphoreType.DMA((2,)),
            pltpu.SemaphoreType.DMA((2,)),
        ],
        compiler_params=pltpu.CompilerParams(
            collective_id=0,
            vmem_limit_bytes=64 * 1024 * 1024,
        ),
    )(x, w_mat, scale_x, scale_w)


# device time: 105159 ns/iter; 1.1050x vs baseline; 1.0065x over previous
import jax
import jax.numpy as jnp
from jax import lax
from jax.experimental import pallas as pl
from jax.experimental.pallas import tpu as pltpu

N_DEV = 4


def kernel(x, w_mat, scale_x, scale_w):
    m_per, k = x.shape
    n_total = w_mat.shape[1]
    n_per = n_total // N_DEV
    H = m_per // 2
    QS = [(o, 128) for o in range(0, 512, 128)]
    KS = 1024
    n_strips = 2 * (k // KS)

    def body(x_ref, w_ref, sx_ref, sw_ref, out_ref, rbuf, hbuf, x8,
             xstrip, w8, wstrip, stg, ssems, rsems, xsems, wsems, osems):
        my = lax.axis_index("i")
        left = (my - 1) % N_DEV
        right = (my + 1) % N_DEV
        opp = (my + 2) % N_DEV
        scale = sx_ref[0] * sw_ref[0]

        def x_dma(h):
            d = pltpu.make_async_copy(
                x_ref.at[pl.ds(h * H, H)], xstrip, xsems.at[0])
            d.start()
            return d

        xd = x_dma(0)

        NH = n_per // 2

        def w_dma(j):
            d = pltpu.make_async_copy(
                w_ref.at[pl.ds((j % 4) * KS, KS),
                         pl.ds(my * n_per + (j // 4) * NH, NH)],
                wstrip.at[j % 2],
                wsems.at[j % 2],
            )
            d.start()
            return d

        wd = {j: w_dma(j) for j in range(2)}

        barrier_sem = pltpu.get_barrier_semaphore()
        for nbr in (left, right):
            pl.semaphore_signal(
                barrier_sem, inc=1,
                device_id=(nbr,), device_id_type=pl.DeviceIdType.MESH,
            )
        pl.semaphore_wait(barrier_sem, 2)

        def rc(src, dst, sem_i, dev):
            return pltpu.make_async_remote_copy(
                src_ref=src, dst_ref=dst,
                send_sem=ssems.at[sem_i], recv_sem=rsems.at[sem_i],
                device_id=(dev,), device_id_type=pl.DeviceIdType.MESH,
            )

        p1r = [rc(x8.at[pl.ds(h * H, H)], rbuf.at[0, pl.ds(h * H, H)],
                  h, right) for h in range(2)]
        p1l = [rc(x8.at[pl.ds(h * H, H)], rbuf.at[1, pl.ds(h * H, H)],
                  2 + h, left) for h in range(2)]
        p2r = [rc(rbuf.at[0, pl.ds(o, r)], hbuf.at[0, pl.ds(o, r)],
                  4 + i, right) for i, (o, r) in enumerate(QS)]
        p2l = [rc(rbuf.at[1, pl.ds(H + o, r)], hbuf.at[1, pl.ds(o, r)],
                  4 + len(QS) + i, left) for i, (o, r) in enumerate(QS)]

        xd.wait()
        x8[pl.ds(0, H), :] = xstrip[...].astype(jnp.float8_e5m2)
        p1r[0].start()
        p1l[0].start()
        xd = x_dma(1)
        xd.wait()
        x8[pl.ds(H, H), :] = xstrip[...].astype(jnp.float8_e5m2)
        p1r[1].start()
        p1l[1].start()

        out_dmas = []

        def gemm(src_ref, src_off, out_off, rows, n0=0, nc=n_per):
            i = len(out_dmas)
            slot = i % 2
            if i >= 2:
                out_dmas[i - 2].wait()
            acc = jnp.dot(src_ref[pl.ds(src_off, rows)],
                          w8[:, pl.ds(n0, nc)],
                          preferred_element_type=jnp.float32)
            stg[slot, pl.ds(0, rows), pl.ds(n0, nc)] = acc * scale
            d = pltpu.make_async_copy(
                stg.at[slot, pl.ds(0, rows), pl.ds(n0, nc)],
                out_ref.at[pl.ds(out_off, rows), pl.ds(n0, nc)],
                osems.at[slot],
            )
            d.start()
            out_dmas.append(d)

        for j in range(n_strips):
            wd[j].wait()
            w8[pl.ds((j % 4) * KS, KS), pl.ds((j // 4) * NH, NH)] = (
                wstrip[j % 2].astype(jnp.float8_e5m2))
            if j + 2 < n_strips:
                wd[j + 2] = w_dma(j + 2)
            if j == 3:
                gemm(x8, 0, my * m_per, H, 0, NH)
                gemm(x8, H, my * m_per + H, H, 0, NH)

        gemm(x8, 0, my * m_per, H, NH, NH)
        gemm(x8, H, my * m_per + H, H, NH, NH)

        p1r[0].wait_recv()
        for r in p2r:
            r.start()
        gemm(rbuf.at[0], 0, left * m_per, H)

        p1l[0].wait_recv()
        gemm(rbuf.at[1], 0, right * m_per, H)

        p1r[1].wait_recv()
        gemm(rbuf.at[0], H, left * m_per + H, H)

        p1l[1].wait_recv()
        for r in p2l:
            r.start()
        gemm(rbuf.at[1], H, right * m_per + H, H)

        for i, (o, r) in enumerate(QS):
            p2r[i].wait_recv()
            gemm(hbuf.at[0], o, opp * m_per + o, r)
            p2l[i].wait_recv()
            gemm(hbuf.at[1], o, opp * m_per + H + o, r)

        out_dmas[-2].wait()
        out_dmas[-1].wait()
        for r in p1r + p1l + p2r + p2l:
            r.wait_send()

    return pl.pallas_call(
        body,
        out_shape=jax.ShapeDtypeStruct((N_DEV * m_per, n_per), jnp.float32),
        in_specs=[
            pl.BlockSpec(memory_space=pl.ANY),
            pl.BlockSpec(memory_space=pl.ANY),
            pl.BlockSpec(memory_space=pltpu.SMEM),
            pl.BlockSpec(memory_space=pltpu.SMEM),
        ],
        out_specs=pl.BlockSpec(memory_space=pl.ANY),
        scratch_shapes=[
            pltpu.VMEM((2, m_per, k), jnp.float8_e5m2),
            pltpu.VMEM((2, H, k), jnp.float8_e5m2),
            pltpu.VMEM((m_per, k), jnp.float8_e5m2),
            pltpu.VMEM((H, k), jnp.float32),
            pltpu.VMEM((k, n_per), jnp.float8_e5m2),
            pltpu.VMEM((2, KS, n_per // 2), jnp.float32),
            pltpu.VMEM((2, H, n_per), jnp.float32),
            pltpu.SemaphoreType.DMA((12,)),
            pltpu.SemaphoreType.DMA((12,)),
            pltpu.SemaphoreType.DMA((1,)),
            pltpu.SemaphoreType.DMA((2,)),
            pltpu.SemaphoreType.DMA((2,)),
        ],
        compiler_params=pltpu.CompilerParams(
            collective_id=0,
            vmem_limit_bytes=64 * 1024 * 1024,
        ),
    )(x, w_mat, scale_x, scale_w)
